# baseline (speedup 1.0000x reference)
import functools

import jax
import jax.numpy as jnp
from jax import lax
from jax.experimental import pallas as pl
from jax.experimental.pallas import tpu as pltpu
from jax.experimental.pallas import tpu_sc as plsc

BATCH = 4096
SEQ = 200
VOCAB = 1000000
EMBED = 64
HIDDEN = 128
NUM_OUT = 3

NC = 2
NS = 16
NW = NC * NS
BPW = BATCH // NW
C0, C1 = 120, 80
LANES = 16
NPAIR = EMBED // (2 * LANES)


def _sc_gather_pool(flat_ids, table16):
    mesh = plsc.VectorSubcoreMesh(core_axis_name="c", subcore_axis_name="s")

    @functools.partial(
        pl.kernel,
        out_type=jax.ShapeDtypeStruct((BATCH, EMBED), jnp.float32),
        mesh=mesh,
        compiler_params=pltpu.CompilerParams(
            use_tc_tiling_on_sc=False, needs_layout_passes=False),
        scratch_types=[
            pltpu.VMEM((BPW * SEQ,), jnp.int32),
            pltpu.VMEM((C0, EMBED), jnp.bfloat16),
            pltpu.VMEM((C1, EMBED), jnp.bfloat16),
            pltpu.VMEM((C0, EMBED), jnp.bfloat16),
            pltpu.VMEM((C1, EMBED), jnp.bfloat16),
            pltpu.VMEM((BPW, EMBED), jnp.float32),
            pltpu.SemaphoreType.DMA,
            pltpu.SemaphoreType.DMA,
        ],
    )
    def k(ids_hbm, tbl_hbm, out_hbm, idx_v, a0, a1, b0, b1, pooled_v,
          sem_a, sem_b):
        wid = lax.axis_index("s") * NC + lax.axis_index("c")
        base = wid * BPW
        pltpu.sync_copy(ids_hbm.at[pl.ds(base * SEQ, BPW * SEQ)], idx_v)

        def start(b, r0, r1, sem):
            off = b * SEQ
            pltpu.make_async_copy(
                tbl_hbm.at[idx_v.at[pl.ds(off, C0)]], r0, sem).start()
            pltpu.make_async_copy(
                tbl_hbm.at[idx_v.at[pl.ds(off + C0, C1)]], r1, sem).start()

        def drain(r0, r1, sem):
            pltpu.make_async_copy(
                tbl_hbm.at[idx_v.at[pl.ds(0, C0)]], r0, sem).wait()
            pltpu.make_async_copy(
                tbl_hbm.at[idx_v.at[pl.ds(0, C1)]], r1, sem).wait()

        def accumulate(b, r0, r1):
            def body0(r, acc):
                new = []
                for d in range(NPAIR):
                    lo, hi = plsc.unpack(
                        r0[r, pl.ds(d * 2 * LANES, 2 * LANES)],
                        format=plsc.PackFormat.INTERLEAVED,
                    )
                    new.append(acc[2 * d] + lo)
                    new.append(acc[2 * d + 1] + hi)
                return tuple(new)

            acc = lax.fori_loop(
                0, C0, body0,
                tuple(jnp.zeros((LANES,), jnp.float32)
                      for _ in range(2 * NPAIR)),
            )

            def body1(r, acc):
                new = []
                for d in range(NPAIR):
                    lo, hi = plsc.unpack(
                        r1[r, pl.ds(d * 2 * LANES, 2 * LANES)],
                        format=plsc.PackFormat.INTERLEAVED,
                    )
                    new.append(acc[2 * d] + lo)
                    new.append(acc[2 * d + 1] + hi)
                return tuple(new)

            acc = lax.fori_loop(0, C1, body1, acc)

            evens = lax.iota(jnp.int32, LANES) * 2
            for d in range(NPAIR):
                colbase = d * 2 * LANES
                plsc.store_scatter(
                    pooled_v.at[b], [colbase + evens], acc[2 * d])
                plsc.store_scatter(
                    pooled_v.at[b], [colbase + evens + 1], acc[2 * d + 1])

        start(0, a0, a1, sem_a)

        @pl.loop(0, BPW, step=2)
        def _(b):
            start(b + 1, b0, b1, sem_b)
            drain(a0, a1, sem_a)
            accumulate(b, a0, a1)

            @pl.when(b + 2 < BPW)
            def _():
                start(b + 2, a0, a1, sem_a)

            drain(b0, b1, sem_b)
            accumulate(b + 1, b0, b1)

        pltpu.sync_copy(pooled_v, out_hbm.at[pl.ds(base, BPW)])

    return k(flat_ids, table16)


def _mlp(pooled, W1, b1, W2, b2):
    BB = 512

    def body(p_ref, w1_ref, b1_ref, w2_ref, b2_ref, o_ref):
        x = p_ref[...] * (1.0 / SEQ)
        h = jnp.dot(x, w1_ref[...], preferred_element_type=jnp.float32)
        h = jnp.maximum(h + b1_ref[...], 0.0)
        o_ref[...] = (
            jnp.dot(h, w2_ref[...], preferred_element_type=jnp.float32)
            + b2_ref[...]
        )

    return pl.pallas_call(
        body,
        grid=(BATCH // BB,),
        in_specs=[
            pl.BlockSpec((BB, EMBED), lambda i: (i, 0)),
            pl.BlockSpec((EMBED, HIDDEN), lambda i: (0, 0)),
            pl.BlockSpec((1, HIDDEN), lambda i: (0, 0)),
            pl.BlockSpec((HIDDEN, NUM_OUT), lambda i: (0, 0)),
            pl.BlockSpec((1, NUM_OUT), lambda i: (0, 0)),
        ],
        out_specs=pl.BlockSpec((BB, NUM_OUT), lambda i: (i, 0)),
        out_shape=jax.ShapeDtypeStruct((BATCH, NUM_OUT), jnp.float32),
    )(pooled, W1, b1.reshape(1, HIDDEN), W2, b2.reshape(1, NUM_OUT))


def kernel(input_ids, embedding, W1, b1, W2, b2):
    flat_ids = input_ids.reshape(-1).astype(jnp.int32)
    table16 = embedding.astype(jnp.bfloat16)
    pooled = _sc_gather_pool(flat_ids, table16)
    return _mlp(pooled, W1, b1, W2, b2)
